# baseline (device time: 77138 ns/iter reference)
import jax
import jax.numpy as jnp
from jax import lax
from jax.experimental import pallas as pl
from jax.experimental.pallas import tpu as pltpu

N_DEV = 8
M_PER = 512
N_COL = 2048
N_HALF = N_COL // 2
N_SUB = 8
W_SUB = N_HALF // N_SUB

PAYLOAD = jnp.bfloat16

STREAMS = (((4, 3, 1), 0), ((1, 3, 4), N_HALF))


def kernel(x, w_mat, scale_x, scale_w):
    def body(x_ref, w_ref, sx_ref, sw_ref, out_ref,
             sbuf_a, rbuf_a, ssem_a, rsem_a,
             sbuf_b, rbuf_b, ssem_b, rsem_b):
        me = lax.axis_index("i")
        bufs = (
            (sbuf_a, rbuf_a, ssem_a, rsem_a),
            (sbuf_b, rbuf_b, ssem_b, rsem_b),
        )

        def partner(mask):
            return jnp.bitwise_xor(me, mask)

        def partial(c, lo):
            xs = x_ref[pl.ds(c * M_PER, M_PER), :].astype(jnp.bfloat16)
            ws = w_ref[:, lo:lo + N_HALF].astype(jnp.bfloat16)
            return jnp.dot(xs, ws, preferred_element_type=jnp.float32)

        def rdma(t, slot, s, mask):
            sbuf, rbuf, ssem, rsem = bufs[t]
            return pltpu.make_async_remote_copy(
                src_ref=sbuf.at[slot, s],
                dst_ref=rbuf.at[slot, s],
                send_sem=ssem.at[slot, s],
                recv_sem=rsem.at[slot, s],
                device_id=(partner(mask),),
                device_id_type=pl.DeviceIdType.MESH,
            )

        barrier_sem = pltpu.get_barrier_semaphore()
        for mask in (1, 3, 4):
            pl.semaphore_signal(
                barrier_sem, inc=1,
                device_id=(partner(mask),), device_id_type=pl.DeviceIdType.MESH,
            )
        pl.semaphore_wait(barrier_sem, 3)

        for k in range(4):
            for t, ((m1, m2, m3), lo) in enumerate(STREAMS):
                e = (m1 ^ m2 ^ m3, m1 ^ m2, m1 ^ m3, m1)[k]
                p = partial(jnp.bitwise_xor(me, e), lo)
                sbuf = bufs[t][0]
                for s in range(N_SUB):
                    sbuf[k, s] = p[:, s * W_SUB:(s + 1) * W_SUB].astype(PAYLOAD)
                    rdma(t, k, s, m1).start()

        for j in range(2):
            for t, ((m1, m2, m3), lo) in enumerate(STREAMS):
                e = (m2 ^ m3, m2)[j]
                p = partial(jnp.bitwise_xor(me, e), lo)
                sbuf, rbuf = bufs[t][0], bufs[t][1]
                for s in range(N_SUB):
                    rdma(t, j, s, m1).wait()
                    acc = rbuf[j, s].astype(jnp.float32) \
                        + p[:, s * W_SUB:(s + 1) * W_SUB]
                    sbuf[4 + j, s] = acc.astype(PAYLOAD)
                    rdma(t, 4 + j, s, m2).start()

        for t, ((m1, m2, m3), lo) in enumerate(STREAMS):
            p = partial(jnp.bitwise_xor(me, m3), lo)
            sbuf, rbuf = bufs[t][0], bufs[t][1]
            for s in range(N_SUB):
                rdma(t, 2, s, m1).wait()
                rdma(t, 4, s, m2).wait()
                acc = rbuf[2, s].astype(jnp.float32) \
                    + rbuf[4, s].astype(jnp.float32) \
                    + p[:, s * W_SUB:(s + 1) * W_SUB]
                sbuf[6, s] = acc.astype(PAYLOAD)
                rdma(t, 6, s, m3).start()

        s_out = sx_ref[0] * sw_ref[0]
        for t, ((m1, m2, m3), lo) in enumerate(STREAMS):
            p = partial(me, lo)
            rbuf = bufs[t][1]
            for s in range(N_SUB):
                rdma(t, 3, s, m1).wait()
                rdma(t, 5, s, m2).wait()
                rdma(t, 6, s, m3).wait()
                acc = rbuf[3, s].astype(jnp.float32) \
                    + rbuf[5, s].astype(jnp.float32) \
                    + rbuf[6, s].astype(jnp.float32) \
                    + p[:, s * W_SUB:(s + 1) * W_SUB]
                out_ref[:, lo + s * W_SUB:lo + (s + 1) * W_SUB] = acc * s_out

    comm = pltpu.VMEM((7, N_SUB, M_PER, W_SUB), PAYLOAD)
    sems = pltpu.SemaphoreType.DMA((7, N_SUB))
    return pl.pallas_call(
        body,
        out_shape=jax.ShapeDtypeStruct((M_PER, N_COL), jnp.float32),
        in_specs=[
            pl.BlockSpec(memory_space=pltpu.VMEM),
            pl.BlockSpec(memory_space=pltpu.VMEM),
            pl.BlockSpec(memory_space=pltpu.SMEM),
            pl.BlockSpec(memory_space=pltpu.SMEM),
        ],
        out_specs=pl.BlockSpec(memory_space=pltpu.VMEM),
        scratch_shapes=[comm, comm, sems, sems, comm, comm, sems, sems],
        compiler_params=pltpu.CompilerParams(
            collective_id=0,
            vmem_limit_bytes=100 * 1024 * 1024,
        ),
    )(x, w_mat, scale_x, scale_w)


# device time: 76909 ns/iter; 1.0030x vs baseline; 1.0030x over previous
import jax
import jax.numpy as jnp
from jax import lax
from jax.experimental import pallas as pl
from jax.experimental.pallas import tpu as pltpu

N_DEV = 8
M_PER = 512
N_COL = 2048
N_HALF = N_COL // 2
N_SUB = 4
W_SUB = N_HALF // N_SUB

PAYLOAD = jnp.bfloat16

STREAMS = (((4, 3, 1), 0), ((1, 3, 4), N_HALF))


def kernel(x, w_mat, scale_x, scale_w):
    def body(x_ref, w_ref, sx_ref, sw_ref, out_ref,
             x_bf, w_bf,
             sbuf_a, rbuf_a, ssem_a, rsem_a,
             sbuf_b, rbuf_b, ssem_b, rsem_b):
        me = lax.axis_index("i")
        bufs = (
            (sbuf_a, rbuf_a, ssem_a, rsem_a),
            (sbuf_b, rbuf_b, ssem_b, rsem_b),
        )

        def partner(mask):
            return jnp.bitwise_xor(me, mask)

        def partial(c, lo):
            xs = x_bf[pl.ds(c * M_PER, M_PER), :]
            ws = w_bf[:, lo:lo + N_HALF]
            return jnp.dot(xs, ws, preferred_element_type=jnp.float32)

        def rdma(t, slot, s, mask):
            sbuf, rbuf, ssem, rsem = bufs[t]
            return pltpu.make_async_remote_copy(
                src_ref=sbuf.at[slot, s],
                dst_ref=rbuf.at[slot, s],
                send_sem=ssem.at[slot, s],
                recv_sem=rsem.at[slot, s],
                device_id=(partner(mask),),
                device_id_type=pl.DeviceIdType.MESH,
            )

        x_bf[...] = x_ref[...].astype(jnp.bfloat16)
        w_bf[...] = w_ref[...].astype(jnp.bfloat16)

        barrier_sem = pltpu.get_barrier_semaphore()
        for mask in (1, 3, 4):
            pl.semaphore_signal(
                barrier_sem, inc=1,
                device_id=(partner(mask),), device_id_type=pl.DeviceIdType.MESH,
            )
        pl.semaphore_wait(barrier_sem, 3)

        for k in range(4):
            for t, ((m1, m2, m3), lo) in enumerate(STREAMS):
                e = (m1 ^ m2 ^ m3, m1 ^ m2, m1 ^ m3, m1)[k]
                p = partial(jnp.bitwise_xor(me, e), lo)
                sbuf = bufs[t][0]
                for s in range(N_SUB):
                    sbuf[k, s] = p[:, s * W_SUB:(s + 1) * W_SUB].astype(PAYLOAD)
                    rdma(t, k, s, m1).start()

        for j in range(2):
            for t, ((m1, m2, m3), lo) in enumerate(STREAMS):
                e = (m2 ^ m3, m2)[j]
                p = partial(jnp.bitwise_xor(me, e), lo)
                sbuf, rbuf = bufs[t][0], bufs[t][1]
                for s in range(N_SUB):
                    rdma(t, j, s, m1).wait()
                    acc = rbuf[j, s].astype(jnp.float32) \
                        + p[:, s * W_SUB:(s + 1) * W_SUB]
                    sbuf[4 + j, s] = acc.astype(PAYLOAD)
                    rdma(t, 4 + j, s, m2).start()

        for t, ((m1, m2, m3), lo) in enumerate(STREAMS):
            p = partial(jnp.bitwise_xor(me, m3), lo)
            sbuf, rbuf = bufs[t][0], bufs[t][1]
            for s in range(N_SUB):
                rdma(t, 2, s, m1).wait()
                rdma(t, 4, s, m2).wait()
                acc = rbuf[2, s].astype(jnp.float32) \
                    + rbuf[4, s].astype(jnp.float32) \
                    + p[:, s * W_SUB:(s + 1) * W_SUB]
                sbuf[6, s] = acc.astype(PAYLOAD)
                rdma(t, 6, s, m3).start()

        s_out = sx_ref[0] * sw_ref[0]
        for t, ((m1, m2, m3), lo) in enumerate(STREAMS):
            p = partial(me, lo)
            rbuf = bufs[t][1]
            for s in range(N_SUB):
                rdma(t, 3, s, m1).wait()
                rdma(t, 5, s, m2).wait()
                rdma(t, 6, s, m3).wait()
                acc = rbuf[3, s].astype(jnp.float32) \
                    + rbuf[5, s].astype(jnp.float32) \
                    + rbuf[6, s].astype(jnp.float32) \
                    + p[:, s * W_SUB:(s + 1) * W_SUB]
                out_ref[:, lo + s * W_SUB:lo + (s + 1) * W_SUB] = acc * s_out

    comm = pltpu.VMEM((7, N_SUB, M_PER, W_SUB), PAYLOAD)
    sems = pltpu.SemaphoreType.DMA((7, N_SUB))
    return pl.pallas_call(
        body,
        out_shape=jax.ShapeDtypeStruct((M_PER, N_COL), jnp.float32),
        in_specs=[
            pl.BlockSpec(memory_space=pltpu.VMEM),
            pl.BlockSpec(memory_space=pltpu.VMEM),
            pl.BlockSpec(memory_space=pltpu.SMEM),
            pl.BlockSpec(memory_space=pltpu.SMEM),
        ],
        out_specs=pl.BlockSpec(memory_space=pltpu.VMEM),
        scratch_shapes=[
            pltpu.VMEM((N_DEV * M_PER, M_PER), jnp.bfloat16),
            pltpu.VMEM((M_PER, N_COL), jnp.bfloat16),
            comm, comm, sems, sems, comm, comm, sems, sems,
        ],
        compiler_params=pltpu.CompilerParams(
            collective_id=0,
            vmem_limit_bytes=100 * 1024 * 1024,
        ),
    )(x, w_mat, scale_x, scale_w)
